# baseline (speedup 1.0000x reference)
import functools

import jax
import jax.numpy as jnp
from jax import lax
from jax.experimental import pallas as pl
from jax.experimental.pallas import tpu as pltpu
from jax.experimental.pallas import tpu_sc as plsc

N = 10000
E = 160000
D = 256
K = 3

NC = 2
NQ = 2
NS = 16
DQ = D // (NC * NQ)
ET = E // NS
CH = 128
NBUF = 3
NCH = (ET + CH - 1) // CH
NCH = ((NCH + NBUF - 1) // NBUF) * NBUF
NP = 10240
RPT = NP // NS


def _make_kernel():
    mesh = plsc.VectorSubcoreMesh(core_axis_name="c", subcore_axis_name="s")

    @functools.partial(
        pl.kernel,
        out_type=jax.ShapeDtypeStruct((NC * NQ * NP, DQ), jnp.float32),
        mesh=mesh,
        compiler_params=pltpu.CompilerParams(use_tc_tiling_on_sc=False),
        scratch_types=(
            [pltpu.VMEM((NCH + NBUF, CH), jnp.int32),
             pltpu.VMEM((NCH, CH), jnp.int32)]
            + [pltpu.VMEM((CH, DQ), jnp.float32)
               for _ in range(NBUF)]
            + [pltpu.VMEM_SHARED((NP, DQ), jnp.float32),
               pltpu.VMEM_SHARED((NP, DQ), jnp.float32)]
            + [pltpu.SemaphoreType.DMA for _ in range(NBUF)]
        ),
    )
    def kgin_sc(xr_hbm, colb_hbm, rowb_hbm, zrows_hbm, out_hbm,
                colv, rowv, *rest):
        bufs = rest[:NBUF]
        ping = rest[NBUF]
        pong = rest[NBUF + 1]
        gsems = rest[NBUF + 2:]
        c = lax.axis_index("c")
        s = lax.axis_index("s")

        pltpu.sync_copy(colb_hbm.at[s], colv)
        pltpu.sync_copy(rowb_hbm.at[s], rowv)

        def edge_loop(src, dst):
            for b in range(NBUF):
                pltpu.async_copy(src.at[colv.at[b]], bufs[b], gsems[b])

            def outer(i, carry):
                j0 = i * NBUF
                for b in range(NBUF):
                    j = j0 + b
                    pltpu.make_async_copy(src.at[colv.at[j]], bufs[b],
                                          gsems[b]).wait()
                    pltpu.sync_copy(bufs[b], dst.at[rowv.at[j]], add=True)
                    pltpu.async_copy(src.at[colv.at[j + NBUF]], bufs[b],
                                     gsems[b])
                return carry

            lax.fori_loop(0, NCH // NBUF, outer, 0)

            for b in range(NBUF):
                pltpu.make_async_copy(src.at[colv.at[0]], bufs[b],
                                      gsems[b]).wait()

        my = pl.ds(s * RPT, RPT)
        for q in range(NQ):
            f = c * NQ + q
            pltpu.sync_copy(xr_hbm.at[pl.ds(f * NP + s * RPT, RPT)],
                            ping.at[my])
            pltpu.sync_copy(zrows_hbm, pong.at[my])
            plsc.subcore_barrier()

            for h in range(K):
                src, dst = (ping, pong) if h % 2 == 0 else (pong, ping)
                edge_loop(src, dst)
                plsc.subcore_barrier()
                if h < K - 1:
                    pltpu.sync_copy(zrows_hbm, src.at[my])
                    plsc.subcore_barrier()
                else:
                    pltpu.sync_copy(dst.at[my],
                                    out_hbm.at[pl.ds(f * NP + s * RPT, RPT)])
                    plsc.subcore_barrier()

    return kgin_sc


_KGIN = _make_kernel()


def kernel(x, adj1):
    row = adj1[0].astype(jnp.int32)
    col = adj1[1].astype(jnp.int32)

    pad = NCH * CH - ET
    col_t = jnp.pad(col.reshape(NS, ET), ((0, 0), (0, pad)))
    row_t = jnp.pad(row.reshape(NS, ET), ((0, 0), (0, pad)),
                    constant_values=N)
    col_t = col_t.reshape(NS, NCH, CH)
    row_t = row_t.reshape(NS, NCH, CH)

    colb = jnp.concatenate(
        [col_t, jnp.zeros((NS, NBUF, CH), jnp.int32)], axis=1)

    xp = jnp.pad(x, ((0, NP - N), (0, 0)))
    xr = xp.reshape(NP, NC * NQ, DQ).transpose(1, 0, 2).reshape(
        NC * NQ * NP, DQ)
    zrows = jnp.zeros((RPT, DQ), jnp.float32)

    out_r = _KGIN(xr, colb, row_t, zrows)
    return out_r.reshape(NC * NQ, NP, DQ)[:, :N].transpose(1, 0, 2).reshape(
        N, D)


# speedup vs baseline: 1.1707x; 1.1707x over previous
import functools

import jax
import jax.numpy as jnp
from jax import lax
from jax.experimental import pallas as pl
from jax.experimental.pallas import tpu as pltpu
from jax.experimental.pallas import tpu_sc as plsc

N = 10000
E = 160000
D = 256
K = 3

NC = 2
NQ = 2
NS = 16
DQ = D // (NC * NQ)
ET = E // NS
CH = 128
NBUF = 2
NCH = (ET + CH - 1) // CH
NCH = ((NCH + NBUF - 1) // NBUF) * NBUF
NP = 10240
RPT = NP // NS


def _make_kernel():
    mesh = plsc.VectorSubcoreMesh(core_axis_name="c", subcore_axis_name="s")

    @functools.partial(
        pl.kernel,
        out_type=jax.ShapeDtypeStruct((N, D), jnp.float32),
        mesh=mesh,
        compiler_params=pltpu.CompilerParams(use_tc_tiling_on_sc=False),
        scratch_types=(
            [pltpu.VMEM((NCH + NBUF, CH), jnp.int32),
             pltpu.VMEM((NCH, CH), jnp.int32)]
            + [pltpu.VMEM((CH, DQ), jnp.float32)
               for _ in range(NBUF)]
            + [pltpu.VMEM_SHARED((NP, DQ), jnp.float32),
               pltpu.VMEM_SHARED((NP, DQ), jnp.float32)]
            + [pltpu.SemaphoreType.DMA for _ in range(NBUF)]
        ),
    )
    def kgin_sc(xr_hbm, colb_hbm, rowb_hbm, zrows_hbm, out_hbm,
                colv, rowv, *rest):
        bufs = rest[:NBUF]
        ping = rest[NBUF]
        pong = rest[NBUF + 1]
        gsems = rest[NBUF + 2:]
        c = lax.axis_index("c")
        s = lax.axis_index("s")

        pltpu.sync_copy(colb_hbm.at[s], colv)
        pltpu.sync_copy(rowb_hbm.at[s], rowv)

        def edge_loop(src, dst):
            for b in range(NBUF):
                pltpu.async_copy(src.at[colv.at[b]], bufs[b], gsems[b])

            def outer(i, carry):
                j0 = i * NBUF
                for b in range(NBUF):
                    j = j0 + b
                    pltpu.make_async_copy(src.at[colv.at[j]], bufs[b],
                                          gsems[b]).wait()
                    pltpu.sync_copy(bufs[b], dst.at[rowv.at[j]], add=True)
                    pltpu.async_copy(src.at[colv.at[j + NBUF]], bufs[b],
                                     gsems[b])
                return carry

            lax.fori_loop(0, NCH // NBUF, outer, 0)

            for b in range(NBUF):
                pltpu.make_async_copy(src.at[colv.at[0]], bufs[b],
                                      gsems[b]).wait()

        my = pl.ds(s * RPT, RPT)
        tail = NS - 1
        trows = N - tail * RPT
        for q in range(NQ):
            fcol = pl.ds((c * NQ + q) * DQ, DQ)
            @pl.when(s < tail)
            def _():
                pltpu.sync_copy(xr_hbm.at[pl.ds(s * RPT, RPT), fcol],
                                ping.at[my])

            @pl.when(s == tail)
            def _():
                pltpu.sync_copy(xr_hbm.at[pl.ds(tail * RPT, trows), fcol],
                                ping.at[pl.ds(tail * RPT, trows)])

            pltpu.sync_copy(zrows_hbm, pong.at[my])
            plsc.subcore_barrier()

            for h in range(K):
                src, dst = (ping, pong) if h % 2 == 0 else (pong, ping)
                edge_loop(src, dst)
                plsc.subcore_barrier()
                if h < K - 1:
                    pltpu.sync_copy(zrows_hbm, src.at[my])
                    plsc.subcore_barrier()
                else:
                    @pl.when(s < tail)
                    def _():
                        pltpu.sync_copy(dst.at[my],
                                        out_hbm.at[pl.ds(s * RPT, RPT), fcol])

                    @pl.when(s == tail)
                    def _():
                        pltpu.sync_copy(
                            dst.at[pl.ds(tail * RPT, trows)],
                            out_hbm.at[pl.ds(tail * RPT, trows), fcol])

                    plsc.subcore_barrier()

    return kgin_sc


_KGIN = _make_kernel()


def kernel(x, adj1):
    row = adj1[0].astype(jnp.int32)
    col = adj1[1].astype(jnp.int32)

    pad = NCH * CH - ET
    col_t = jnp.pad(col.reshape(NS, ET), ((0, 0), (0, pad)))
    row_t = jnp.pad(row.reshape(NS, ET), ((0, 0), (0, pad)),
                    constant_values=N)
    col_t = col_t.reshape(NS, NCH, CH)
    row_t = row_t.reshape(NS, NCH, CH)

    colb = jnp.concatenate(
        [col_t, jnp.zeros((NS, NBUF, CH), jnp.int32)], axis=1)

    zrows = jnp.zeros((RPT, DQ), jnp.float32)
    return _KGIN(x, colb, row_t, zrows)


# speedup vs baseline: 1.1866x; 1.0136x over previous
import functools

import jax
import jax.numpy as jnp
from jax import lax
from jax.experimental import pallas as pl
from jax.experimental.pallas import tpu as pltpu
from jax.experimental.pallas import tpu_sc as plsc

N = 10000
E = 160000
D = 256
K = 3

NC = 2
NQ = 2
NS = 16
DQ = D // (NC * NQ)
ET = E // NS
CH = 128
NBUF = 2
NCH = (ET + CH - 1) // CH
NCH = ((NCH + NBUF - 1) // NBUF) * NBUF
NP = 10240
RPT = NP // NS


def _make_kernel():
    mesh = plsc.VectorSubcoreMesh(core_axis_name="c", subcore_axis_name="s")

    @functools.partial(
        pl.kernel,
        out_type=jax.ShapeDtypeStruct((N, D), jnp.float32),
        mesh=mesh,
        compiler_params=pltpu.CompilerParams(use_tc_tiling_on_sc=False),
        scratch_types=(
            [pltpu.VMEM((NCH + NBUF, CH), jnp.int32),
             pltpu.VMEM((NCH, CH), jnp.int32)]
            + [pltpu.VMEM((CH, DQ), jnp.float32)
               for _ in range(NBUF)]
            + [pltpu.VMEM_SHARED((NP, DQ), jnp.float32),
               pltpu.VMEM_SHARED((NP, DQ), jnp.float32)]
            + [pltpu.SemaphoreType.DMA for _ in range(NBUF)]
        ),
    )
    def kgin_sc(xr_hbm, colb_hbm, rowb_hbm, zrows_hbm, out_hbm,
                colv, rowv, *rest):
        bufs = rest[:NBUF]
        ping = rest[NBUF]
        pong = rest[NBUF + 1]
        gsems = rest[NBUF + 2:]
        c = lax.axis_index("c")
        s = lax.axis_index("s")

        pltpu.sync_copy(colb_hbm.at[s], colv)
        pltpu.sync_copy(rowb_hbm.at[s], rowv)

        def edge_loop(src, dst):
            for b in range(NBUF):
                pltpu.async_copy(src.at[colv.at[b]], bufs[b], gsems[b])

            def outer(i, carry):
                j0 = i * NBUF
                for b in range(NBUF):
                    j = j0 + b
                    pltpu.make_async_copy(src.at[colv.at[j]], bufs[b],
                                          gsems[b]).wait()
                    pltpu.sync_copy(bufs[b], dst.at[rowv.at[j]], add=True)
                    pltpu.async_copy(src.at[colv.at[j + NBUF]], bufs[b],
                                     gsems[b])
                return carry

            lax.fori_loop(0, NCH // NBUF, outer, 0)

            for b in range(NBUF):
                pltpu.make_async_copy(src.at[colv.at[0]], bufs[b],
                                      gsems[b]).wait()

        my = pl.ds(s * RPT, RPT)
        tail = NS - 1
        trows = N - tail * RPT

        def stage(q):
            fcol = pl.ds((c * NQ + q) * DQ, DQ)

            @pl.when(s < tail)
            def _():
                pltpu.async_copy(xr_hbm.at[pl.ds(s * RPT, RPT), fcol],
                                 ping.at[my], gsems[0])

            @pl.when(s == tail)
            def _():
                pltpu.async_copy(xr_hbm.at[pl.ds(tail * RPT, trows), fcol],
                                 ping.at[pl.ds(tail * RPT, trows)], gsems[0])

        def wait_stage():
            @pl.when(s < tail)
            def _():
                pltpu.make_async_copy(zrows_hbm, ping.at[my],
                                      gsems[0]).wait()

            @pl.when(s == tail)
            def _():
                pltpu.make_async_copy(zrows_hbm.at[pl.ds(0, trows)],
                                      ping.at[pl.ds(tail * RPT, trows)],
                                      gsems[0]).wait()

        def publish(q, dst):
            fcol = pl.ds((c * NQ + q) * DQ, DQ)

            @pl.when(s < tail)
            def _():
                pltpu.sync_copy(dst.at[my],
                                out_hbm.at[pl.ds(s * RPT, RPT), fcol])

            @pl.when(s == tail)
            def _():
                pltpu.sync_copy(dst.at[pl.ds(tail * RPT, trows)],
                                out_hbm.at[pl.ds(tail * RPT, trows), fcol])

        stage(0)
        pltpu.sync_copy(zrows_hbm, pong.at[my])
        wait_stage()
        plsc.subcore_barrier()

        for q in range(NQ):
            for h in range(K):
                src, dst = (ping, pong) if h % 2 == 0 else (pong, ping)
                edge_loop(src, dst)
                plsc.subcore_barrier()
                if h < K - 1:
                    pltpu.sync_copy(zrows_hbm, src.at[my])
                    plsc.subcore_barrier()
            if q < NQ - 1:
                stage(q + 1)
                publish(q, pong)
                pltpu.sync_copy(zrows_hbm, pong.at[my])
                wait_stage()
                plsc.subcore_barrier()
            else:
                publish(q, pong)
                plsc.subcore_barrier()

    return kgin_sc


_KGIN = _make_kernel()


def kernel(x, adj1):
    row = adj1[0].astype(jnp.int32)
    col = adj1[1].astype(jnp.int32)

    pad = NCH * CH - ET
    col_t = jnp.pad(col.reshape(NS, ET), ((0, 0), (0, pad)))
    row_t = jnp.pad(row.reshape(NS, ET), ((0, 0), (0, pad)),
                    constant_values=N)
    col_t = col_t.reshape(NS, NCH, CH)
    row_t = row_t.reshape(NS, NCH, CH)

    colb = jnp.concatenate(
        [col_t, jnp.zeros((NS, NBUF, CH), jnp.int32)], axis=1)

    zrows = jnp.zeros((RPT, DQ), jnp.float32)
    return _KGIN(x, colb, row_t, zrows)
